# baseline (speedup 1.0000x reference)
import functools

import jax
import jax.numpy as jnp
from jax import lax
from jax.experimental import pallas as pl
from jax.experimental.pallas import tpu as pltpu
from jax.experimental.pallas import tpu_sc as plsc

N = 10000
E = 320000
D = 128
H = 128
C = 2
G = 64

NC = 2
NS = 16
NW = NC * NS
EPW = E // NW
CH = 80
NCHUNK = EPW // CH

HW = H // NC
CH2 = 128
NCHUNK2 = -(-E // (NS * CH2))
EPAD = NS * CH2 * NCHUNK2
DUMP = N
ACCR = N + 8
RPW = 624
TAIL = N - NS * RPW
ZR = 104

DEGW = 624
DEGT = N - NS * DEGW

_mesh = plsc.VectorSubcoreMesh(
    core_axis_name="c", subcore_axis_name="s", num_cores=NC, num_subcores=NS
)


def _zero_vmem_2d(ref, rows, cols):
    def body(i, _):
        r = i // (cols // 16)
        j = (i % (cols // 16)) * 16
        ref[r, pl.ds(j, 16)] = jnp.zeros((16,), jnp.float32)
        return 0
    lax.fori_loop(0, rows * (cols // 16), body, 0)


def _zero_vmem_1d(ref, n):
    def body(i, _):
        ref[pl.ds(i * 16, 16)] = jnp.zeros((16,), jnp.float32)
        return 0
    lax.fori_loop(0, n // 16, body, 0)


@functools.partial(
    pl.kernel,
    out_type=jax.ShapeDtypeStruct((NC * N,), jnp.float32),
    mesh=_mesh,
    scratch_types=[
        pltpu.VMEM((NCHUNK, CH), jnp.int32),
        pltpu.VMEM((CH,), jnp.float32),
        pltpu.VMEM((DEGW,), jnp.float32),
        pltpu.VMEM_SHARED((N,), jnp.float32),
    ],
)
def _sc_degree(dst_hbm, out_hbm, didx_v, ones_v, zbuf_v, dacc):
    c = lax.axis_index("c")
    s = lax.axis_index("s")
    w = c * NS + s

    _zero_vmem_1d(zbuf_v, DEGW)
    def setones(i, _):
        ones_v[pl.ds(i * 16, 16)] = jnp.ones((16,), jnp.float32)
        return 0
    lax.fori_loop(0, CH // 16, setones, 0)

    pltpu.sync_copy(zbuf_v, dacc.at[pl.ds(s * DEGW, DEGW)])
    @pl.when(s == 0)
    def _():
        pltpu.sync_copy(zbuf_v.at[pl.ds(0, DEGT)], dacc.at[pl.ds(NS * DEGW, DEGT)])
    plsc.subcore_barrier()

    pltpu.sync_copy(dst_hbm.at[w], didx_v)

    def body(i, _):
        pltpu.sync_copy(ones_v, dacc.at[didx_v.at[i]], add=True)
        return 0
    lax.fori_loop(0, NCHUNK, body, 0)

    plsc.subcore_barrier()
    pltpu.sync_copy(dacc.at[pl.ds(s * DEGW, DEGW)], zbuf_v)
    pltpu.sync_copy(zbuf_v, out_hbm.at[pl.ds(c * N + s * DEGW, DEGW)])
    @pl.when(s == 0)
    def _():
        pltpu.sync_copy(dacc.at[pl.ds(NS * DEGW, DEGT)], zbuf_v.at[pl.ds(0, DEGT)])
        pltpu.sync_copy(zbuf_v.at[pl.ds(0, DEGT)],
                        out_hbm.at[pl.ds(c * N + NS * DEGW, DEGT)])


K = 5


@functools.partial(
    pl.kernel,
    out_type=jax.ShapeDtypeStruct((NC, N, HW), jnp.float32),
    mesh=_mesh,
    compiler_params=pltpu.CompilerParams(use_tc_tiling_on_sc=False),
    scratch_types=(
        [
            pltpu.VMEM((NCHUNK2, CH2), jnp.int32),
            pltpu.VMEM((NCHUNK2, CH2), jnp.int32),
        ]
        + [pltpu.VMEM((CH2, HW), jnp.float32) for _ in range(K)]
        + [
            pltpu.VMEM((ZR, HW), jnp.float32),
            pltpu.VMEM_SHARED((ACCR, HW), jnp.float32),
        ]
        + [pltpu.SemaphoreType.DMA for _ in range(2 * K)]
    ),
)
def _sc_scatter(z_hbm, src_hbm, dst_hbm, out_hbm, sidx_v, didx_v, *rest):
    rows = rest[:K]
    zbuf_v = rest[K]
    acc = rest[K + 1]
    gsem = rest[K + 2:2 * K + 2]
    ssem = rest[2 * K + 2:3 * K + 2]
    c = lax.axis_index("c")
    s = lax.axis_index("s")
    row0 = s * RPW

    _zero_vmem_2d(zbuf_v, ZR, HW)
    for t in range(RPW // ZR):
        pltpu.sync_copy(zbuf_v, acc.at[pl.ds(row0 + t * ZR, ZR)])
    @pl.when(s == 0)
    def _():
        pltpu.sync_copy(zbuf_v.at[pl.ds(0, TAIL)], acc.at[pl.ds(NS * RPW, TAIL)])

    pltpu.sync_copy(src_hbm.at[s], sidx_v)
    pltpu.sync_copy(dst_hbm.at[s], didx_v)
    @pl.when(c == 1)
    def _():
        def adj(i, _):
            r = i // (CH2 // 16)
            j = (i % (CH2 // 16)) * 16
            sidx_v[r, pl.ds(j, 16)] = sidx_v[r, pl.ds(j, 16)] + N
            return 0
        lax.fori_loop(0, NCHUNK2 * (CH2 // 16), adj, 0)
    plsc.subcore_barrier()

    for p in range(K - 1):
        pltpu.async_copy(z_hbm.at[sidx_v.at[p]], rows[p], gsem[p])

    def body(i, _):
        for m in range(K):
            mp = (m + K - 1) % K
            @pl.when(i % K == m)
            def _(m=m, mp=mp):
                pltpu.make_async_copy(z_hbm.at[sidx_v.at[i]], rows[m], gsem[m]).wait()
                pltpu.async_copy(rows[m], acc.at[didx_v.at[i]], ssem[m], add=True)
                @pl.when(i + K - 1 < NCHUNK2)
                def _():
                    @pl.when(i >= 1)
                    def _():
                        pltpu.make_async_copy(
                            rows[mp], acc.at[didx_v.at[i - 1]], ssem[mp]).wait()
                    pltpu.async_copy(z_hbm.at[sidx_v.at[i + K - 1]], rows[mp], gsem[mp])
        return 0
    lax.fori_loop(0, NCHUNK2, body, 0)

    for j in range(NCHUNK2 - K, NCHUNK2):
        pltpu.make_async_copy(rows[j % K], acc.at[didx_v.at[j]], ssem[j % K]).wait()

    plsc.subcore_barrier()
    for t in range(RPW // ZR):
        pltpu.sync_copy(acc.at[pl.ds(row0 + t * ZR, ZR)], zbuf_v)
        pltpu.sync_copy(zbuf_v, out_hbm.at[c, pl.ds(row0 + t * ZR, ZR)])
    @pl.when(s == 0)
    def _():
        pltpu.sync_copy(acc.at[pl.ds(NS * RPW, TAIL)], zbuf_v.at[pl.ds(0, TAIL)])
        pltpu.sync_copy(zbuf_v.at[pl.ds(0, TAIL)],
                        out_hbm.at[c, pl.ds(NS * RPW, TAIL)])


_NCHIP = 5
_RC = N // _NCHIP


def _tc_prep_body(deg_ref, x_ref, xn_ref, dinv_ref):
    deg = deg_ref[0] + deg_ref[1] + 1.0
    dinv = lax.rsqrt(deg)
    xn = x_ref[...] * dinv
    xn_ref[0] = xn[:, :HW]
    xn_ref[1] = xn[:, HW:]
    dinv_ref[...] = dinv


def _tc_mid_body(acc_ref, z_ref, dinv_ref, b_ref, w_ref, batch_ref, wfc_ref,
                 bfc_ref, zn_ref, out_ref, pool_acc, cnt_acc):
    i = pl.program_id(0)

    @pl.when(i == 0)
    def _():
        pool_acc[...] = jnp.zeros_like(pool_acc)
        cnt_acc[...] = jnp.zeros_like(cnt_acc)

    dinv = dinv_ref[...]
    u = (jnp.concatenate([acc_ref[0], acc_ref[1]], axis=1)
         + jnp.concatenate([z_ref[0], z_ref[1]], axis=1))
    v = jnp.dot(u, w_ref[...], preferred_element_type=jnp.float32)
    h = jnp.maximum(v * dinv + b_ref[...], 0.0)
    zn = h * dinv
    zn_ref[0] = zn[:, :HW]
    zn_ref[1] = zn[:, HW:]

    seg = jax.lax.broadcasted_iota(jnp.int32, (G, _RC), 0)
    oh = (seg == batch_ref[0]).astype(jnp.float32)
    pool_acc[...] += jnp.dot(oh, h, preferred_element_type=jnp.float32)
    cnt_acc[...] += jnp.sum(oh, axis=1, keepdims=True)

    @pl.when(i == _NCHIP - 1)
    def _():
        pooled = pool_acc[...] / jnp.maximum(cnt_acc[...], 1.0)
        logits = jnp.dot(pooled, wfc_ref[...], preferred_element_type=jnp.float32)
        logits = logits + bfc_ref[...]
        m = jnp.max(logits, axis=1, keepdims=True)
        z = logits - m
        out_ref[...] = z - jnp.log(jnp.sum(jnp.exp(z), axis=1, keepdims=True))


def _row_chunk(i):
    return (i, 0)


_tc_prep = pl.pallas_call(
    _tc_prep_body,
    grid=(_NCHIP,),
    in_specs=[
        pl.BlockSpec((NC, _RC, 1), lambda i: (0, i, 0)),
        pl.BlockSpec((_RC, D), _row_chunk),
    ],
    out_specs=[
        pl.BlockSpec((NC, _RC, HW), lambda i: (0, i, 0)),
        pl.BlockSpec((_RC, 1), _row_chunk),
    ],
    out_shape=[
        jax.ShapeDtypeStruct((NC, N, HW), jnp.float32),
        jax.ShapeDtypeStruct((N, 1), jnp.float32),
    ],
)

_tc_mid = pl.pallas_call(
    _tc_mid_body,
    grid=(_NCHIP,),
    in_specs=[
        pl.BlockSpec((NC, _RC, HW), lambda i: (0, i, 0)),
        pl.BlockSpec((NC, _RC, HW), lambda i: (0, i, 0)),
        pl.BlockSpec((_RC, 1), _row_chunk),
        pl.BlockSpec((1, H), lambda i: (0, 0)),
        pl.BlockSpec((H, H), lambda i: (0, 0)),
        pl.BlockSpec((1, 1, _RC), lambda i: (i, 0, 0)),
        pl.BlockSpec((H, C), lambda i: (0, 0)),
        pl.BlockSpec((1, C), lambda i: (0, 0)),
    ],
    out_specs=[
        pl.BlockSpec((NC, _RC, HW), lambda i: (0, i, 0)),
        pl.BlockSpec((G, C), lambda i: (0, 0)),
    ],
    out_shape=[
        jax.ShapeDtypeStruct((NC, N, HW), jnp.float32),
        jax.ShapeDtypeStruct((G, C), jnp.float32),
    ],
    scratch_shapes=[
        pltpu.VMEM((G, H), jnp.float32),
        pltpu.VMEM((G, 1), jnp.float32),
    ],
)


def kernel(x, edge_index, batch, W1, b1, W2, b2, Wfc, bfc):
    dst_deg = edge_index[1].reshape(NW, NCHUNK, CH)
    npad = EPAD - E
    pad_src = jnp.arange(npad, dtype=jnp.int32) % 16
    pad_dst = jnp.full((npad,), DUMP, dtype=jnp.int32)
    src2 = jnp.concatenate([edge_index[0], pad_src]).reshape(NS, NCHUNK2, CH2)
    dst2 = jnp.concatenate([edge_index[1], pad_dst]).reshape(NS, NCHUNK2, CH2)
    batch2 = batch.reshape(_NCHIP, 1, _RC)

    deg = _sc_degree(dst_deg)
    deg3 = deg.reshape(NC, N, 1)

    xn, dinv = _tc_prep(deg3, x)

    bs = jnp.stack([b1, b2]).reshape(2, 1, H)
    Ws = jnp.stack([W1, W2])

    def step(z, xs):
        b_i, w_i = xs
        acc = _sc_scatter(z.reshape(NC * N, HW), src2, dst2)
        zn, logits = _tc_mid(acc, z, dinv, b_i, w_i, batch2,
                             Wfc, bfc.reshape(1, C))
        return zn, logits

    _, outs = lax.scan(step, xn, (bs, Ws))
    return outs[1]


# speedup vs baseline: 1.0642x; 1.0642x over previous
import functools

import jax
import jax.numpy as jnp
from jax import lax
from jax.experimental import pallas as pl
from jax.experimental.pallas import tpu as pltpu
from jax.experimental.pallas import tpu_sc as plsc

N = 10000
E = 320000
D = 128
H = 128
C = 2
G = 64

NC = 2
NS = 16
NW = NC * NS
EPW = E // NW
CH = 80
NCHUNK = EPW // CH

HW = H // NC
CH2 = 128
NCHUNK2 = -(-E // (NS * CH2))
EPAD = NS * CH2 * NCHUNK2
DUMP = N
ACCR = N + 8
RPW = 624
TAIL = N - NS * RPW
ZR = 104

DEGW = 624
DEGT = N - NS * DEGW

_mesh = plsc.VectorSubcoreMesh(
    core_axis_name="c", subcore_axis_name="s", num_cores=NC, num_subcores=NS
)


def _zero_vmem_2d(ref, rows, cols):
    def body(i, _):
        r = i // (cols // 16)
        j = (i % (cols // 16)) * 16
        ref[r, pl.ds(j, 16)] = jnp.zeros((16,), jnp.float32)
        return 0
    lax.fori_loop(0, rows * (cols // 16), body, 0)


def _zero_vmem_1d(ref, n):
    def body(i, _):
        ref[pl.ds(i * 16, 16)] = jnp.zeros((16,), jnp.float32)
        return 0
    lax.fori_loop(0, n // 16, body, 0)


@functools.partial(
    pl.kernel,
    out_type=jax.ShapeDtypeStruct((NC * N,), jnp.float32),
    mesh=_mesh,
    scratch_types=[
        pltpu.VMEM((NCHUNK, CH), jnp.int32),
        pltpu.VMEM((CH,), jnp.float32),
        pltpu.VMEM((DEGW,), jnp.float32),
        pltpu.VMEM_SHARED((N,), jnp.float32),
        pltpu.SemaphoreType.DMA,
    ],
)
def _sc_degree(dst_hbm, out_hbm, didx_v, ones_v, zbuf_v, dacc, dsem):
    c = lax.axis_index("c")
    s = lax.axis_index("s")
    w = c * NS + s

    _zero_vmem_1d(zbuf_v, DEGW)
    def setones(i, _):
        ones_v[pl.ds(i * 16, 16)] = jnp.ones((16,), jnp.float32)
        return 0
    lax.fori_loop(0, CH // 16, setones, 0)

    pltpu.sync_copy(zbuf_v, dacc.at[pl.ds(s * DEGW, DEGW)])
    @pl.when(s == 0)
    def _():
        pltpu.sync_copy(zbuf_v.at[pl.ds(0, DEGT)], dacc.at[pl.ds(NS * DEGW, DEGT)])
    plsc.subcore_barrier()

    pltpu.sync_copy(dst_hbm.at[w], didx_v)

    def body(i, _):
        pltpu.async_copy(ones_v, dacc.at[didx_v.at[i]], dsem, add=True)
        return 0
    lax.fori_loop(0, NCHUNK, body, 0)

    def drain(i, _):
        pltpu.make_async_copy(ones_v, dacc.at[didx_v.at[i]], dsem).wait()
        return 0
    lax.fori_loop(0, NCHUNK, drain, 0)

    plsc.subcore_barrier()
    pltpu.sync_copy(dacc.at[pl.ds(s * DEGW, DEGW)], zbuf_v)
    pltpu.sync_copy(zbuf_v, out_hbm.at[pl.ds(c * N + s * DEGW, DEGW)])
    @pl.when(s == 0)
    def _():
        pltpu.sync_copy(dacc.at[pl.ds(NS * DEGW, DEGT)], zbuf_v.at[pl.ds(0, DEGT)])
        pltpu.sync_copy(zbuf_v.at[pl.ds(0, DEGT)],
                        out_hbm.at[pl.ds(c * N + NS * DEGW, DEGT)])


K = 5


@functools.partial(
    pl.kernel,
    out_type=jax.ShapeDtypeStruct((NC, N, HW), jnp.float32),
    mesh=_mesh,
    compiler_params=pltpu.CompilerParams(use_tc_tiling_on_sc=False),
    scratch_types=(
        [
            pltpu.VMEM((NCHUNK2, CH2), jnp.int32),
            pltpu.VMEM((NCHUNK2, CH2), jnp.int32),
        ]
        + [pltpu.VMEM((CH2, HW), jnp.float32) for _ in range(K)]
        + [
            pltpu.VMEM((ZR, HW), jnp.float32),
            pltpu.VMEM_SHARED((ACCR, HW), jnp.float32),
        ]
        + [pltpu.SemaphoreType.DMA for _ in range(2 * K)]
    ),
)
def _sc_scatter(y_hbm, src_hbm, dst_hbm, out_hbm, sidx_v, didx_v, *rest):
    rows = rest[:K]
    zbuf_v = rest[K]
    acc = rest[K + 1]
    gsem = rest[K + 2:2 * K + 2]
    ssem = rest[2 * K + 2:3 * K + 2]
    c = lax.axis_index("c")
    s = lax.axis_index("s")
    row0 = s * RPW

    _zero_vmem_2d(zbuf_v, ZR, HW)
    for t in range(RPW // ZR):
        pltpu.sync_copy(zbuf_v, acc.at[pl.ds(row0 + t * ZR, ZR)])
    @pl.when(s == 0)
    def _():
        pltpu.sync_copy(zbuf_v.at[pl.ds(0, TAIL)], acc.at[pl.ds(NS * RPW, TAIL)])

    pltpu.sync_copy(src_hbm.at[s], sidx_v)
    pltpu.sync_copy(dst_hbm.at[s], didx_v)
    @pl.when(c == 1)
    def _():
        def adj(i, _):
            r = i // (CH2 // 16)
            j = (i % (CH2 // 16)) * 16
            sidx_v[r, pl.ds(j, 16)] = sidx_v[r, pl.ds(j, 16)] + N
            return 0
        lax.fori_loop(0, NCHUNK2 * (CH2 // 16), adj, 0)
    plsc.subcore_barrier()

    for p in range(K - 1):
        pltpu.async_copy(y_hbm.at[sidx_v.at[p]], rows[p], gsem[p])

    def body(i, _):
        for m in range(K):
            mp = (m + K - 1) % K
            @pl.when(i % K == m)
            def _(m=m, mp=mp):
                pltpu.make_async_copy(y_hbm.at[sidx_v.at[i]], rows[m], gsem[m]).wait()
                pltpu.async_copy(rows[m], acc.at[didx_v.at[i]], ssem[m], add=True)
                @pl.when(i + K - 1 < NCHUNK2)
                def _():
                    @pl.when(i >= 1)
                    def _():
                        pltpu.make_async_copy(
                            rows[mp], acc.at[didx_v.at[i - 1]], ssem[mp]).wait()
                    pltpu.async_copy(y_hbm.at[sidx_v.at[i + K - 1]], rows[mp], gsem[mp])
        return 0
    lax.fori_loop(0, NCHUNK2, body, 0)

    for j in range(NCHUNK2 - K, NCHUNK2):
        pltpu.make_async_copy(rows[j % K], acc.at[didx_v.at[j]], ssem[j % K]).wait()

    plsc.subcore_barrier()
    for t in range(RPW // ZR):
        pltpu.sync_copy(acc.at[pl.ds(row0 + t * ZR, ZR)], zbuf_v)
        pltpu.sync_copy(zbuf_v, out_hbm.at[c, pl.ds(row0 + t * ZR, ZR)])
    @pl.when(s == 0)
    def _():
        pltpu.sync_copy(acc.at[pl.ds(NS * RPW, TAIL)], zbuf_v.at[pl.ds(0, TAIL)])
        pltpu.sync_copy(zbuf_v.at[pl.ds(0, TAIL)],
                        out_hbm.at[c, pl.ds(NS * RPW, TAIL)])


_NCHIP = 5
_RC = N // _NCHIP


def _tc_prep_body(deg_ref, x_ref, w1_ref, xw_ref, y_ref, dinv_ref):
    deg = deg_ref[0] + deg_ref[1] + 1.0
    dinv = lax.rsqrt(deg)
    xw = jnp.dot(x_ref[...], w1_ref[...], preferred_element_type=jnp.float32)
    xw_ref[...] = xw
    y = xw * dinv
    y_ref[0] = y[:, :HW]
    y_ref[1] = y[:, HW:]
    dinv_ref[...] = dinv


def _tc_mid_body(acc_ref, xw_ref, dinv_ref, b_ref, wn_ref, batch_ref, wfc_ref,
                 bfc_ref, xwn_ref, yn_ref, out_ref, pool_acc, cnt_acc):
    i = pl.program_id(0)

    @pl.when(i == 0)
    def _():
        pool_acc[...] = jnp.zeros_like(pool_acc)
        cnt_acc[...] = jnp.zeros_like(cnt_acc)

    dinv = dinv_ref[...]
    agg = jnp.concatenate([acc_ref[0], acc_ref[1]], axis=1)
    h = jnp.maximum(agg * dinv + xw_ref[...] * (dinv * dinv) + b_ref[...], 0.0)
    xwn = jnp.dot(h, wn_ref[...], preferred_element_type=jnp.float32)
    xwn_ref[...] = xwn
    yn = xwn * dinv
    yn_ref[0] = yn[:, :HW]
    yn_ref[1] = yn[:, HW:]

    seg = jax.lax.broadcasted_iota(jnp.int32, (G, _RC), 0)
    oh = (seg == batch_ref[0]).astype(jnp.float32)
    pool_acc[...] += jnp.dot(oh, h, preferred_element_type=jnp.float32)
    cnt_acc[...] += jnp.sum(oh, axis=1, keepdims=True)

    @pl.when(i == _NCHIP - 1)
    def _():
        pooled = pool_acc[...] / jnp.maximum(cnt_acc[...], 1.0)
        logits = jnp.dot(pooled, wfc_ref[...], preferred_element_type=jnp.float32)
        logits = logits + bfc_ref[...]
        m = jnp.max(logits, axis=1, keepdims=True)
        z = logits - m
        out_ref[...] = z - jnp.log(jnp.sum(jnp.exp(z), axis=1, keepdims=True))


def _row_chunk(i):
    return (i, 0)


_tc_prep = pl.pallas_call(
    _tc_prep_body,
    grid=(_NCHIP,),
    in_specs=[
        pl.BlockSpec((NC, _RC, 1), lambda i: (0, i, 0)),
        pl.BlockSpec((_RC, D), _row_chunk),
        pl.BlockSpec((D, H), lambda i: (0, 0)),
    ],
    out_specs=[
        pl.BlockSpec((_RC, H), _row_chunk),
        pl.BlockSpec((NC, _RC, HW), lambda i: (0, i, 0)),
        pl.BlockSpec((_RC, 1), _row_chunk),
    ],
    out_shape=[
        jax.ShapeDtypeStruct((N, H), jnp.float32),
        jax.ShapeDtypeStruct((NC, N, HW), jnp.float32),
        jax.ShapeDtypeStruct((N, 1), jnp.float32),
    ],
)

_tc_mid = pl.pallas_call(
    _tc_mid_body,
    grid=(_NCHIP,),
    in_specs=[
        pl.BlockSpec((NC, _RC, HW), lambda i: (0, i, 0)),
        pl.BlockSpec((_RC, H), _row_chunk),
        pl.BlockSpec((_RC, 1), _row_chunk),
        pl.BlockSpec((1, H), lambda i: (0, 0)),
        pl.BlockSpec((H, H), lambda i: (0, 0)),
        pl.BlockSpec((1, 1, _RC), lambda i: (i, 0, 0)),
        pl.BlockSpec((H, C), lambda i: (0, 0)),
        pl.BlockSpec((1, C), lambda i: (0, 0)),
    ],
    out_specs=[
        pl.BlockSpec((_RC, H), _row_chunk),
        pl.BlockSpec((NC, _RC, HW), lambda i: (0, i, 0)),
        pl.BlockSpec((G, C), lambda i: (0, 0)),
    ],
    out_shape=[
        jax.ShapeDtypeStruct((N, H), jnp.float32),
        jax.ShapeDtypeStruct((NC, N, HW), jnp.float32),
        jax.ShapeDtypeStruct((G, C), jnp.float32),
    ],
    scratch_shapes=[
        pltpu.VMEM((G, H), jnp.float32),
        pltpu.VMEM((G, 1), jnp.float32),
    ],
)


def kernel(x, edge_index, batch, W1, b1, W2, b2, Wfc, bfc):
    dst_deg = edge_index[1].reshape(NW, NCHUNK, CH)
    npad = EPAD - E
    pad_src = jnp.arange(npad, dtype=jnp.int32) % 16
    pad_dst = jnp.full((npad,), DUMP, dtype=jnp.int32)
    src2 = jnp.concatenate([edge_index[0], pad_src]).reshape(NS, NCHUNK2, CH2)
    dst2 = jnp.concatenate([edge_index[1], pad_dst]).reshape(NS, NCHUNK2, CH2)
    batch2 = batch.reshape(_NCHIP, 1, _RC)

    deg = _sc_degree(dst_deg)
    deg3 = deg.reshape(NC, N, 1)

    xw1, y1, dinv = _tc_prep(deg3, x, W1)

    bs = jnp.stack([b1, b2]).reshape(2, 1, H)
    Wn = jnp.stack([W2, W2])

    def step(carry, xs):
        xw, y = carry
        b_i, wn_i = xs
        acc = _sc_scatter(y.reshape(NC * N, HW), src2, dst2)
        xwn, yn, logits = _tc_mid(acc, xw, dinv, b_i, wn_i, batch2,
                                  Wfc, bfc.reshape(1, C))
        return (xwn, yn), logits

    _, outs = lax.scan(step, (xw1, y1), (bs, Wn))
    return outs[1]


# speedup vs baseline: 1.1293x; 1.0612x over previous
import functools

import jax
import jax.numpy as jnp
from jax import lax
from jax.experimental import pallas as pl
from jax.experimental.pallas import tpu as pltpu
from jax.experimental.pallas import tpu_sc as plsc

N = 10000
E = 320000
D = 128
H = 128
C = 2
G = 64

NC = 2
NS = 16
NW = NC * NS
EPW = E // NW
CH = 80
NCHUNK = EPW // CH

HW = H // NC
CH2 = 128
NCHUNK2 = -(-E // (NS * CH2))
EPAD = NS * CH2 * NCHUNK2
DUMP = N
ACCR = N + 8
RPW = 624
TAIL = N - NS * RPW
ZR = 104

DEGW = 624
DEGT = N - NS * DEGW

_mesh = plsc.VectorSubcoreMesh(
    core_axis_name="c", subcore_axis_name="s", num_cores=NC, num_subcores=NS
)


def _zero_vmem_2d(ref, rows, cols):
    def body(i, _):
        r = i // (cols // 16)
        j = (i % (cols // 16)) * 16
        ref[r, pl.ds(j, 16)] = jnp.zeros((16,), jnp.float32)
        return 0
    lax.fori_loop(0, rows * (cols // 16), body, 0)


def _zero_vmem_1d(ref, n):
    def body(i, _):
        ref[pl.ds(i * 16, 16)] = jnp.zeros((16,), jnp.float32)
        return 0
    lax.fori_loop(0, n // 16, body, 0)


@functools.partial(
    pl.kernel,
    out_type=jax.ShapeDtypeStruct((NC * N,), jnp.float32),
    mesh=_mesh,
    scratch_types=[
        pltpu.VMEM((NCHUNK, CH), jnp.int32),
        pltpu.VMEM((CH,), jnp.float32),
        pltpu.VMEM((DEGW,), jnp.float32),
        pltpu.VMEM_SHARED((N,), jnp.float32),
        pltpu.SemaphoreType.DMA,
    ],
)
def _sc_degree(dst_hbm, out_hbm, didx_v, ones_v, zbuf_v, dacc, dsem):
    c = lax.axis_index("c")
    s = lax.axis_index("s")
    w = c * NS + s

    _zero_vmem_1d(zbuf_v, DEGW)
    def setones(i, _):
        ones_v[pl.ds(i * 16, 16)] = jnp.ones((16,), jnp.float32)
        return 0
    lax.fori_loop(0, CH // 16, setones, 0)

    pltpu.sync_copy(zbuf_v, dacc.at[pl.ds(s * DEGW, DEGW)])
    @pl.when(s == 0)
    def _():
        pltpu.sync_copy(zbuf_v.at[pl.ds(0, DEGT)], dacc.at[pl.ds(NS * DEGW, DEGT)])
    plsc.subcore_barrier()

    pltpu.sync_copy(dst_hbm.at[w], didx_v)

    def body(i, _):
        pltpu.async_copy(ones_v, dacc.at[didx_v.at[i]], dsem, add=True)
        return 0
    lax.fori_loop(0, NCHUNK, body, 0)

    def drain(i, _):
        pltpu.make_async_copy(ones_v, dacc.at[didx_v.at[i]], dsem).wait()
        return 0
    lax.fori_loop(0, NCHUNK, drain, 0)

    plsc.subcore_barrier()
    pltpu.sync_copy(dacc.at[pl.ds(s * DEGW, DEGW)], zbuf_v)
    pltpu.sync_copy(zbuf_v, out_hbm.at[pl.ds(c * N + s * DEGW, DEGW)])
    @pl.when(s == 0)
    def _():
        pltpu.sync_copy(dacc.at[pl.ds(NS * DEGW, DEGT)], zbuf_v.at[pl.ds(0, DEGT)])
        pltpu.sync_copy(zbuf_v.at[pl.ds(0, DEGT)],
                        out_hbm.at[pl.ds(c * N + NS * DEGW, DEGT)])


K = 5


@functools.partial(
    pl.kernel,
    out_type=jax.ShapeDtypeStruct((NC, N, HW), jnp.float32),
    mesh=_mesh,
    compiler_params=pltpu.CompilerParams(use_tc_tiling_on_sc=False),
    scratch_types=(
        [
            pltpu.VMEM((NCHUNK2, CH2), jnp.int32),
            pltpu.VMEM((NCHUNK2, CH2), jnp.int32),
        ]
        + [pltpu.VMEM((CH2, HW), jnp.float32) for _ in range(K)]
        + [
            pltpu.VMEM((ZR, HW), jnp.float32),
            pltpu.VMEM_SHARED((ACCR, HW), jnp.float32),
        ]
        + [pltpu.SemaphoreType.DMA for _ in range(2 * K)]
    ),
)
def _sc_scatter(y_hbm, src_hbm, dst_hbm, out_hbm, sidx_v, didx_v, *rest):
    rows = rest[:K]
    zbuf_v = rest[K]
    acc = rest[K + 1]
    gsem = rest[K + 2:2 * K + 2]
    ssem = rest[2 * K + 2:3 * K + 2]
    c = lax.axis_index("c")
    s = lax.axis_index("s")
    w = c * NS + s
    row0 = s * RPW

    pltpu.async_copy(src_hbm.at[w], sidx_v, gsem[0])
    pltpu.async_copy(dst_hbm.at[s], didx_v, gsem[1])

    _zero_vmem_2d(zbuf_v, ZR, HW)
    for t in range(RPW // ZR):
        pltpu.async_copy(zbuf_v, acc.at[pl.ds(row0 + t * ZR, ZR)], ssem[0])
    @pl.when(s == 0)
    def _():
        pltpu.sync_copy(zbuf_v.at[pl.ds(0, TAIL)], acc.at[pl.ds(NS * RPW, TAIL)])
    for t in range(RPW // ZR):
        pltpu.make_async_copy(zbuf_v, acc.at[pl.ds(row0 + t * ZR, ZR)],
                              ssem[0]).wait()

    pltpu.make_async_copy(src_hbm.at[w], sidx_v, gsem[0]).wait()
    pltpu.make_async_copy(dst_hbm.at[s], didx_v, gsem[1]).wait()
    plsc.subcore_barrier()

    for p in range(K - 1):
        pltpu.async_copy(y_hbm.at[sidx_v.at[p]], rows[p], gsem[p])

    def body(i, _):
        for m in range(K):
            mp = (m + K - 1) % K
            @pl.when(i % K == m)
            def _(m=m, mp=mp):
                pltpu.make_async_copy(y_hbm.at[sidx_v.at[i]], rows[m], gsem[m]).wait()
                pltpu.async_copy(rows[m], acc.at[didx_v.at[i]], ssem[m], add=True)
                @pl.when(i + K - 1 < NCHUNK2)
                def _():
                    @pl.when(i >= 1)
                    def _():
                        pltpu.make_async_copy(
                            rows[mp], acc.at[didx_v.at[i - 1]], ssem[mp]).wait()
                    pltpu.async_copy(y_hbm.at[sidx_v.at[i + K - 1]], rows[mp], gsem[mp])
        return 0
    lax.fori_loop(0, NCHUNK2, body, 0)

    for j in range(NCHUNK2 - K, NCHUNK2):
        pltpu.make_async_copy(rows[j % K], acc.at[didx_v.at[j]], ssem[j % K]).wait()

    plsc.subcore_barrier()
    wbufs = (zbuf_v, rows[0].at[pl.ds(0, ZR)])
    NT = RPW // ZR
    for t in range(NT):
        b = wbufs[t % 2]
        if t >= 2:
            pltpu.make_async_copy(
                b, out_hbm.at[c, pl.ds(row0 + (t - 2) * ZR, ZR)],
                ssem[1 + t % 2]).wait()
        pltpu.sync_copy(acc.at[pl.ds(row0 + t * ZR, ZR)], b)
        pltpu.async_copy(b, out_hbm.at[c, pl.ds(row0 + t * ZR, ZR)],
                         ssem[1 + t % 2])
    for t in range(NT - 2, NT):
        pltpu.make_async_copy(
            wbufs[t % 2], out_hbm.at[c, pl.ds(row0 + t * ZR, ZR)],
            ssem[1 + t % 2]).wait()
    @pl.when(s == 0)
    def _():
        pltpu.sync_copy(acc.at[pl.ds(NS * RPW, TAIL)], zbuf_v.at[pl.ds(0, TAIL)])
        pltpu.sync_copy(zbuf_v.at[pl.ds(0, TAIL)],
                        out_hbm.at[c, pl.ds(NS * RPW, TAIL)])


_NCHIP = 5
_RC = N // _NCHIP


def _tc_prep_body(deg_ref, x_ref, w1_ref, xw_ref, y_ref, dinv_ref):
    deg = deg_ref[0] + deg_ref[1] + 1.0
    dinv = lax.rsqrt(deg)
    xw = jnp.dot(x_ref[...], w1_ref[...], preferred_element_type=jnp.float32)
    xw_ref[...] = xw
    y = xw * dinv
    y_ref[0] = y[:, :HW]
    y_ref[1] = y[:, HW:]
    dinv_ref[...] = dinv


def _tc_mid_body(acc_ref, xw_ref, dinv_ref, b_ref, wn_ref, batch_ref, wfc_ref,
                 bfc_ref, xwn_ref, yn_ref, out_ref, pool_acc, cnt_acc):
    i = pl.program_id(0)

    @pl.when(i == 0)
    def _():
        pool_acc[...] = jnp.zeros_like(pool_acc)
        cnt_acc[...] = jnp.zeros_like(cnt_acc)

    dinv = dinv_ref[...]
    agg = jnp.concatenate([acc_ref[0], acc_ref[1]], axis=1)
    h = jnp.maximum(agg * dinv + xw_ref[...] * (dinv * dinv) + b_ref[...], 0.0)
    xwn = jnp.dot(h, wn_ref[...], preferred_element_type=jnp.float32)
    xwn_ref[...] = xwn
    yn = xwn * dinv
    yn_ref[0] = yn[:, :HW]
    yn_ref[1] = yn[:, HW:]

    seg = jax.lax.broadcasted_iota(jnp.int32, (G, _RC), 0)
    oh = (seg == batch_ref[0]).astype(jnp.float32)
    pool_acc[...] += jnp.dot(oh, h, preferred_element_type=jnp.float32)
    cnt_acc[...] += jnp.sum(oh, axis=1, keepdims=True)

    @pl.when(i == _NCHIP - 1)
    def _():
        pooled = pool_acc[...] / jnp.maximum(cnt_acc[...], 1.0)
        logits = jnp.dot(pooled, wfc_ref[...], preferred_element_type=jnp.float32)
        logits = logits + bfc_ref[...]
        m = jnp.max(logits, axis=1, keepdims=True)
        z = logits - m
        out_ref[...] = z - jnp.log(jnp.sum(jnp.exp(z), axis=1, keepdims=True))


def _row_chunk(i):
    return (i, 0)


_tc_prep = pl.pallas_call(
    _tc_prep_body,
    grid=(_NCHIP,),
    in_specs=[
        pl.BlockSpec((NC, _RC, 1), lambda i: (0, i, 0)),
        pl.BlockSpec((_RC, D), _row_chunk),
        pl.BlockSpec((D, H), lambda i: (0, 0)),
    ],
    out_specs=[
        pl.BlockSpec((_RC, H), _row_chunk),
        pl.BlockSpec((NC, _RC, HW), lambda i: (0, i, 0)),
        pl.BlockSpec((_RC, 1), _row_chunk),
    ],
    out_shape=[
        jax.ShapeDtypeStruct((N, H), jnp.float32),
        jax.ShapeDtypeStruct((NC, N, HW), jnp.float32),
        jax.ShapeDtypeStruct((N, 1), jnp.float32),
    ],
)

_tc_mid = pl.pallas_call(
    _tc_mid_body,
    grid=(_NCHIP,),
    in_specs=[
        pl.BlockSpec((NC, _RC, HW), lambda i: (0, i, 0)),
        pl.BlockSpec((_RC, H), _row_chunk),
        pl.BlockSpec((_RC, 1), _row_chunk),
        pl.BlockSpec((1, H), lambda i: (0, 0)),
        pl.BlockSpec((H, H), lambda i: (0, 0)),
        pl.BlockSpec((1, 1, _RC), lambda i: (i, 0, 0)),
        pl.BlockSpec((H, C), lambda i: (0, 0)),
        pl.BlockSpec((1, C), lambda i: (0, 0)),
    ],
    out_specs=[
        pl.BlockSpec((_RC, H), _row_chunk),
        pl.BlockSpec((NC, _RC, HW), lambda i: (0, i, 0)),
        pl.BlockSpec((G, C), lambda i: (0, 0)),
    ],
    out_shape=[
        jax.ShapeDtypeStruct((N, H), jnp.float32),
        jax.ShapeDtypeStruct((NC, N, HW), jnp.float32),
        jax.ShapeDtypeStruct((G, C), jnp.float32),
    ],
    scratch_shapes=[
        pltpu.VMEM((G, H), jnp.float32),
        pltpu.VMEM((G, 1), jnp.float32),
    ],
)


def kernel(x, edge_index, batch, W1, b1, W2, b2, Wfc, bfc):
    dst_deg = edge_index[1].reshape(NW, NCHUNK, CH)
    npad = EPAD - E
    pad_src = jnp.arange(npad, dtype=jnp.int32) % 16
    pad_dst = jnp.full((npad,), DUMP, dtype=jnp.int32)
    src_w = jnp.concatenate([edge_index[0], pad_src]).reshape(NS, NCHUNK2, CH2)
    src2 = jnp.concatenate([src_w[None], src_w[None] + N], axis=0)
    src2 = src2.reshape(NW, NCHUNK2, CH2)
    dst2 = jnp.concatenate([edge_index[1], pad_dst]).reshape(NS, NCHUNK2, CH2)
    batch2 = batch.reshape(_NCHIP, 1, _RC)

    deg = _sc_degree(dst_deg)
    deg3 = deg.reshape(NC, N, 1)

    xw1, y1, dinv = _tc_prep(deg3, x, W1)

    bs = jnp.stack([b1, b2]).reshape(2, 1, H)
    Wn = jnp.stack([W2, W2])

    def step(carry, xs):
        xw, y = carry
        b_i, wn_i = xs
        acc = _sc_scatter(y.reshape(NC * N, HW), src2, dst2)
        xwn, yn, logits = _tc_mid(acc, xw, dinv, b_i, wn_i, batch2,
                                  Wfc, bfc.reshape(1, C))
        return (xwn, yn), logits

    _, outs = lax.scan(step, (xw1, y1), (bs, Wn))
    return outs[1]
